# Initial kernel scaffold; baseline (speedup 1.0000x reference)
import jax
import jax.numpy as jnp
from jax.experimental import pallas as pl


def kernel(xyz, new_xyz, points):
    raise NotImplementedError("write your pallas kernel here")


# speedup vs baseline: 11.2269x; 11.2269x over previous
import functools

import jax
import jax.numpy as jnp
from jax import lax
from jax.experimental import pallas as pl
from jax.experimental.pallas import tpu as pltpu
from jax.experimental.pallas import tpu_sc as plsc

NSAMPLE = 32
QT = 256
D_PAD = 80
NW = 32
CHUNK = 128


def _ballquery_body(nq_ref, xyz_ref, out_ref):
    b = pl.program_id(0)
    n = xyz_ref.shape[2]
    qx = nq_ref[0, :, 0:1]
    qy = nq_ref[0, :, 1:2]
    qz = nq_ref[0, :, 2:3]
    px = xyz_ref[0, 0:1, :]
    py = xyz_ref[0, 1:2, :]
    pz = xyz_ref[0, 2:3, :]
    dx = qx - px
    dy = qy - py
    dz = qz - pz
    dist2 = dx * dx + dy * dy + dz * dz
    ar = lax.broadcasted_iota(jnp.int32, (QT, n), 1)
    scores = jnp.where(dist2 < jnp.float32(0.1 * 0.1), ar, n)
    lane = lax.broadcasted_iota(jnp.int32, (QT, NSAMPLE), 1)

    def body(s, carry):
        m, acc = carry
        cand = jnp.where(scores > m, scores, n)
        m2 = jnp.min(cand, axis=1, keepdims=True)
        acc2 = jnp.where(lane == s, m2, acc)
        return m2, acc2

    m0 = jnp.full((QT, 1), -1, jnp.int32)
    acc0 = jnp.zeros((QT, NSAMPLE), jnp.int32)
    _, acc = lax.fori_loop(0, NSAMPLE, body, (m0, acc0))
    first = acc[:, 0:1]
    first = jnp.where(first < n, first, 0)
    idx = jnp.where(acc < n, acc, first)
    out_ref[0, 0] = idx + b * n


def _sc_gather_body(idx_hbm, table_hbm, out_hbm, idx_v, rows_v, sem):
    wid = lax.axis_index("s") * 2 + lax.axis_index("c")
    rows_per_worker = out_hbm.shape[0] // NW
    base = wid * rows_per_worker

    def chunk(j, carry):
        off = base + j * CHUNK
        pltpu.sync_copy(idx_hbm.at[pl.ds(off, CHUNK)], idx_v)
        pltpu.async_copy(table_hbm.at[idx_v], rows_v, sem).wait()
        pltpu.sync_copy(rows_v, out_hbm.at[pl.ds(off, CHUNK), :])
        return carry

    lax.fori_loop(0, rows_per_worker // CHUNK, chunk, 0)


def kernel(xyz, new_xyz, points):
    B, N, _ = xyz.shape
    npoint = new_xyz.shape[1]
    C = points.shape[-1]
    nt = npoint // QT

    xyz_t = jnp.transpose(xyz, (0, 2, 1))
    flat_idx = pl.pallas_call(
        _ballquery_body,
        grid=(B, nt),
        in_specs=[
            pl.BlockSpec((1, QT, 3), lambda b, t: (b, t, 0)),
            pl.BlockSpec((1, 3, N), lambda b, t: (b, 0, 0)),
        ],
        out_specs=pl.BlockSpec((1, 1, QT, NSAMPLE), lambda b, t: (b, t, 0, 0)),
        out_shape=jax.ShapeDtypeStruct((B, nt, QT, NSAMPLE), jnp.int32),
    )(new_xyz, xyz_t).reshape(-1)

    pad = jnp.zeros((B, N, D_PAD - 3 - C), jnp.float32)
    table = jnp.concatenate([xyz, points, pad], axis=-1).reshape(B * N, D_PAD)

    rows = B * npoint * NSAMPLE
    mesh = plsc.VectorSubcoreMesh(core_axis_name="c", subcore_axis_name="s")
    out80 = pl.kernel(
        _sc_gather_body,
        out_type=jax.ShapeDtypeStruct((rows, D_PAD), jnp.float32),
        mesh=mesh,
        scratch_types=[
            pltpu.VMEM((CHUNK,), jnp.int32),
            pltpu.VMEM((CHUNK, D_PAD), jnp.float32),
            pltpu.SemaphoreType.DMA,
        ],
        compiler_params=pltpu.CompilerParams(use_tc_tiling_on_sc=False),
    )(flat_idx, table)

    out80 = out80.reshape(B, npoint, NSAMPLE, D_PAD)
    grouped_xyz = out80[..., 0:3] - new_xyz[:, :, None, :]
    return jnp.concatenate([grouped_xyz, out80[..., 3 : 3 + C]], axis=-1)


# speedup vs baseline: 14.2393x; 1.2683x over previous
import functools

import jax
import jax.numpy as jnp
from jax import lax
from jax.experimental import pallas as pl
from jax.experimental.pallas import tpu as pltpu
from jax.experimental.pallas import tpu_sc as plsc

NSAMPLE = 32
QT = 256
D_PAD = 80
NW = 32
CHUNK = 128


def _ballquery_body(nq_ref, xyz_ref, out_ref):
    b = pl.program_id(0)
    n = xyz_ref.shape[2]
    qx = nq_ref[0, :, 0:1]
    qy = nq_ref[0, :, 1:2]
    qz = nq_ref[0, :, 2:3]
    px = xyz_ref[0, 0:1, :]
    py = xyz_ref[0, 1:2, :]
    pz = xyz_ref[0, 2:3, :]
    dx = qx - px
    dy = qy - py
    dz = qz - pz
    dist2 = dx * dx + dy * dy + dz * dz
    sbit = jnp.int32(-(2**31))
    ar_x = lax.broadcasted_iota(jnp.int32, (QT, n), 1) + sbit
    n32 = jnp.int32(n)
    scores_x = jnp.where(dist2 < jnp.float32(0.1 * 0.1), ar_x, n32 + sbit)
    lane = lax.broadcasted_iota(jnp.int32, (QT, NSAMPLE), 1)

    def body(s, carry):
        m1, acc = carry
        keymin = jnp.min(scores_x - m1, axis=1, keepdims=True)
        thr = (n32 + sbit) - m1
        m2 = jnp.where(keymin <= thr, keymin + m1 + sbit, n32)
        acc2 = jnp.where(lane == s, m2, acc)
        return jnp.minimum(m2 + 1, n32), acc2

    m0 = jnp.zeros((QT, 1), jnp.int32)
    acc0 = jnp.zeros((QT, NSAMPLE), jnp.int32)
    _, acc = lax.fori_loop(0, NSAMPLE, body, (m0, acc0))
    first = acc[:, 0:1]
    first = jnp.where(first < n, first, 0)
    idx = jnp.where(acc < n, acc, first)
    out_ref[0, 0] = idx + b * n


def _sc_gather_body(idx_hbm, table_hbm, out_hbm, idx_v, rows_v, sem):
    wid = lax.axis_index("s") * 2 + lax.axis_index("c")
    rows_per_worker = out_hbm.shape[0] // NW
    base = wid * rows_per_worker

    def chunk(j, carry):
        off = base + j * CHUNK
        pltpu.sync_copy(idx_hbm.at[pl.ds(off, CHUNK)], idx_v)
        pltpu.async_copy(table_hbm.at[idx_v], rows_v, sem).wait()
        pltpu.sync_copy(rows_v, out_hbm.at[pl.ds(off, CHUNK), :])
        return carry

    lax.fori_loop(0, rows_per_worker // CHUNK, chunk, 0)


def kernel(xyz, new_xyz, points):
    B, N, _ = xyz.shape
    npoint = new_xyz.shape[1]
    C = points.shape[-1]
    nt = npoint // QT

    xyz_t = jnp.transpose(xyz, (0, 2, 1))
    flat_idx = pl.pallas_call(
        _ballquery_body,
        grid=(B, nt),
        in_specs=[
            pl.BlockSpec((1, QT, 3), lambda b, t: (b, t, 0)),
            pl.BlockSpec((1, 3, N), lambda b, t: (b, 0, 0)),
        ],
        out_specs=pl.BlockSpec((1, 1, QT, NSAMPLE), lambda b, t: (b, t, 0, 0)),
        out_shape=jax.ShapeDtypeStruct((B, nt, QT, NSAMPLE), jnp.int32),
    )(new_xyz, xyz_t).reshape(-1)

    pad = jnp.zeros((B, N, D_PAD - 3 - C), jnp.float32)
    table = jnp.concatenate([xyz, points, pad], axis=-1).reshape(B * N, D_PAD)

    rows = B * npoint * NSAMPLE
    mesh = plsc.VectorSubcoreMesh(core_axis_name="c", subcore_axis_name="s")
    out80 = pl.kernel(
        _sc_gather_body,
        out_type=jax.ShapeDtypeStruct((rows, D_PAD), jnp.float32),
        mesh=mesh,
        scratch_types=[
            pltpu.VMEM((CHUNK,), jnp.int32),
            pltpu.VMEM((CHUNK, D_PAD), jnp.float32),
            pltpu.SemaphoreType.DMA,
        ],
        compiler_params=pltpu.CompilerParams(use_tc_tiling_on_sc=False),
    )(flat_idx, table)

    out80 = out80.reshape(B, npoint, NSAMPLE, D_PAD)
    grouped_xyz = out80[..., 0:3] - new_xyz[:, :, None, :]
    return jnp.concatenate([grouped_xyz, out80[..., 3 : 3 + C]], axis=-1)


# speedup vs baseline: 14.8056x; 1.0398x over previous
import functools

import jax
import jax.numpy as jnp
from jax import lax
from jax.experimental import pallas as pl
from jax.experimental.pallas import tpu as pltpu
from jax.experimental.pallas import tpu_sc as plsc

NSAMPLE = 32
QT = 256
NW = 32
CHUNK = 128


def _ballquery_body(nq_ref, xyz_ref, out_ref):
    b = pl.program_id(0)
    n = xyz_ref.shape[2]
    qx = nq_ref[0, :, 0:1]
    qy = nq_ref[0, :, 1:2]
    qz = nq_ref[0, :, 2:3]
    px = xyz_ref[0, 0:1, :]
    py = xyz_ref[0, 1:2, :]
    pz = xyz_ref[0, 2:3, :]
    dx = qx - px
    dy = qy - py
    dz = qz - pz
    dist2 = dx * dx + dy * dy + dz * dz
    sbit = jnp.int32(-(2**31))
    ar_x = lax.broadcasted_iota(jnp.int32, (QT, n), 1) + sbit
    n32 = jnp.int32(n)
    scores_x = jnp.where(dist2 < jnp.float32(0.1 * 0.1), ar_x, n32 + sbit)
    lane = lax.broadcasted_iota(jnp.int32, (QT, NSAMPLE), 1)

    def body(s, carry):
        m1, acc = carry
        keymin = jnp.min(scores_x - m1, axis=1, keepdims=True)
        thr = (n32 + sbit) - m1
        m2 = jnp.where(keymin <= thr, keymin + m1 + sbit, n32)
        acc2 = jnp.where(lane == s, m2, acc)
        return jnp.minimum(m2 + 1, n32), acc2

    m0 = jnp.zeros((QT, 1), jnp.int32)
    acc0 = jnp.zeros((QT, NSAMPLE), jnp.int32)
    _, acc = lax.fori_loop(0, NSAMPLE, body, (m0, acc0))
    first = acc[:, 0:1]
    first = jnp.where(first < n, first, 0)
    idx = jnp.where(acc < n, acc, first)
    out_ref[0, 0] = idx + b * n


def _sc_gather_body(idx_hbm, table_hbm, out_hbm,
                    idx_all, rows_a, rows_b, sem_a, sem_b):
    wid = lax.axis_index("s") * 2 + lax.axis_index("c")
    rows_per_worker = out_hbm.shape[0] // NW
    nch = rows_per_worker // CHUNK
    base = wid * rows_per_worker

    pltpu.sync_copy(idx_hbm.at[pl.ds(wid * nch, nch)], idx_all)

    def pair(i, carry):
        a = 2 * i
        b = 2 * i + 1
        ha = pltpu.async_copy(table_hbm.at[idx_all.at[a]], rows_a, sem_a)
        hb = pltpu.async_copy(table_hbm.at[idx_all.at[b]], rows_b, sem_b)
        ha.wait()
        pltpu.sync_copy(rows_a, out_hbm.at[pl.ds(base + a * CHUNK, CHUNK), :])
        hb.wait()
        pltpu.sync_copy(rows_b, out_hbm.at[pl.ds(base + b * CHUNK, CHUNK), :])
        return carry

    lax.fori_loop(0, nch // 2, pair, 0)


def kernel(xyz, new_xyz, points):
    B, N, _ = xyz.shape
    npoint = new_xyz.shape[1]
    C = points.shape[-1]
    nt = npoint // QT

    xyz_t = jnp.transpose(xyz, (0, 2, 1))
    flat_idx = pl.pallas_call(
        _ballquery_body,
        grid=(B, nt),
        in_specs=[
            pl.BlockSpec((1, QT, 3), lambda b, t: (b, t, 0)),
            pl.BlockSpec((1, 3, N), lambda b, t: (b, 0, 0)),
        ],
        out_specs=pl.BlockSpec((1, 1, QT, NSAMPLE), lambda b, t: (b, t, 0, 0)),
        out_shape=jax.ShapeDtypeStruct((B, nt, QT, NSAMPLE), jnp.int32),
    )(new_xyz, xyz_t).reshape(-1)

    d_tab = 80
    padt = jnp.zeros((B, N, d_tab - 3 - C), jnp.float32)
    table = jnp.concatenate([xyz, points, padt], axis=-1).reshape(B * N, d_tab)

    rows = B * npoint * NSAMPLE
    nch = rows // NW // CHUNK
    mesh = plsc.VectorSubcoreMesh(core_axis_name="c", subcore_axis_name="s")
    out = pl.kernel(
        _sc_gather_body,
        out_type=jax.ShapeDtypeStruct((rows, d_tab), jnp.float32),
        mesh=mesh,
        scratch_types=[
            pltpu.VMEM((nch, CHUNK), jnp.int32),
            pltpu.VMEM((CHUNK, d_tab), jnp.float32),
            pltpu.VMEM((CHUNK, d_tab), jnp.float32),
            pltpu.SemaphoreType.DMA,
            pltpu.SemaphoreType.DMA,
        ],
        compiler_params=pltpu.CompilerParams(use_tc_tiling_on_sc=False),
    )(flat_idx.reshape(rows // CHUNK, CHUNK), table)

    out = out.reshape(B, npoint, NSAMPLE, d_tab)
    grouped_xyz = out[..., 0:3] - new_xyz[:, :, None, :]
    return jnp.concatenate([grouped_xyz, out[..., 3 : 3 + C]], axis=-1)


# speedup vs baseline: 15.3701x; 1.0381x over previous
import functools

import jax
import jax.numpy as jnp
from jax import lax
from jax.experimental import pallas as pl
from jax.experimental.pallas import tpu as pltpu
from jax.experimental.pallas import tpu_sc as plsc

NSAMPLE = 32
QT = 256
NW = 32
CHUNK = 128


def _ballquery_body(nq_ref, xyz_ref, out_ref):
    b = pl.program_id(0)
    n = xyz_ref.shape[2]
    qx = nq_ref[0, :, 0:1]
    qy = nq_ref[0, :, 1:2]
    qz = nq_ref[0, :, 2:3]
    px = xyz_ref[0, 0:1, :]
    py = xyz_ref[0, 1:2, :]
    pz = xyz_ref[0, 2:3, :]
    dx = qx - px
    dy = qy - py
    dz = qz - pz
    dist2 = dx * dx + dy * dy + dz * dz
    sbit = jnp.int32(-(2**31))
    ar_x = lax.broadcasted_iota(jnp.int32, (QT, n), 1) + sbit
    n32 = jnp.int32(n)
    scores_x = jnp.where(dist2 < jnp.float32(0.1 * 0.1), ar_x, n32 + sbit)
    lane = lax.broadcasted_iota(jnp.int32, (QT, NSAMPLE), 1)

    def body(s, carry):
        m1, acc = carry
        keymin = jnp.min(scores_x - m1, axis=1, keepdims=True)
        thr = (n32 + sbit) - m1
        m2 = jnp.where(keymin <= thr, keymin + m1 + sbit, n32)
        acc2 = jnp.where(lane == s, m2, acc)
        return jnp.minimum(m2 + 1, n32), acc2

    m0 = jnp.zeros((QT, 1), jnp.int32)
    acc0 = jnp.zeros((QT, NSAMPLE), jnp.int32)
    _, acc = lax.fori_loop(0, NSAMPLE, body, (m0, acc0))
    first = acc[:, 0:1]
    first = jnp.where(first < n, first, 0)
    idx = jnp.where(acc < n, acc, first)
    out_ref[0, 0] = idx + b * n


def _sc_gather_body(idx_hbm, table_hbm, out_hbm,
                    idx_all, rows_a, rows_b, sem_a, sem_b):
    wid = lax.axis_index("s") * 2 + lax.axis_index("c")
    rows_per_worker = out_hbm.shape[0] // NW
    nch = rows_per_worker // CHUNK
    base = wid * rows_per_worker

    pltpu.sync_copy(idx_hbm.at[pl.ds(wid * nch, nch)], idx_all)

    def pair(i, carry):
        a = 2 * i
        b = 2 * i + 1
        ha = pltpu.async_copy(table_hbm.at[idx_all.at[a]], rows_a, sem_a)
        hb = pltpu.async_copy(table_hbm.at[idx_all.at[b]], rows_b, sem_b)
        ha.wait()
        pltpu.sync_copy(rows_a, out_hbm.at[pl.ds(base + a * CHUNK, CHUNK), :])
        hb.wait()
        pltpu.sync_copy(rows_b, out_hbm.at[pl.ds(base + b * CHUNK, CHUNK), :])
        return carry

    lax.fori_loop(0, nch // 2, pair, 0)


def _split_pipeline(xyz, new_xyz, points):
    B, N, _ = xyz.shape
    npoint = new_xyz.shape[1]
    C = points.shape[-1]
    nt = npoint // QT

    xyz_t = jnp.transpose(xyz, (0, 2, 1))
    flat_idx = pl.pallas_call(
        _ballquery_body,
        grid=(B, nt),
        in_specs=[
            pl.BlockSpec((1, QT, 3), lambda b, t: (b, t, 0)),
            pl.BlockSpec((1, 3, N), lambda b, t: (b, 0, 0)),
        ],
        out_specs=pl.BlockSpec((1, 1, QT, NSAMPLE), lambda b, t: (b, t, 0, 0)),
        out_shape=jax.ShapeDtypeStruct((B, nt, QT, NSAMPLE), jnp.int32),
    )(new_xyz, xyz_t).reshape(-1)

    d_tab = 80
    padt = jnp.zeros((B, N, d_tab - 3 - C), jnp.float32)
    table = jnp.concatenate([xyz, points, padt], axis=-1).reshape(B * N, d_tab)

    rows = B * npoint * NSAMPLE
    nch = rows // NW // CHUNK
    mesh = plsc.VectorSubcoreMesh(core_axis_name="c", subcore_axis_name="s")
    out = pl.kernel(
        _sc_gather_body,
        out_type=jax.ShapeDtypeStruct((rows, d_tab), jnp.float32),
        mesh=mesh,
        scratch_types=[
            pltpu.VMEM((nch, CHUNK), jnp.int32),
            pltpu.VMEM((CHUNK, d_tab), jnp.float32),
            pltpu.VMEM((CHUNK, d_tab), jnp.float32),
            pltpu.SemaphoreType.DMA,
            pltpu.SemaphoreType.DMA,
        ],
        compiler_params=pltpu.CompilerParams(use_tc_tiling_on_sc=False),
    )(flat_idx.reshape(rows // CHUNK, CHUNK), table)

    out = out.reshape(B, npoint, NSAMPLE, d_tab)
    grouped_xyz = out[..., 0:3] - new_xyz[:, :, None, :]
    return jnp.concatenate([grouped_xyz, out[..., 3 : 3 + C]], axis=-1)


def kernel(xyz, new_xyz, points):
    B = xyz.shape[0]
    h = B // 2
    lo = _split_pipeline(xyz[:h], new_xyz[:h], points[:h])
    hi = _split_pipeline(xyz[h:], new_xyz[h:], points[h:])
    return jnp.concatenate([lo, hi], axis=0)


# speedup vs baseline: 15.4405x; 1.0046x over previous
import functools

import jax
import jax.numpy as jnp
from jax import lax
from jax.experimental import pallas as pl
from jax.experimental.pallas import tpu as pltpu
from jax.experimental.pallas import tpu_sc as plsc

NSAMPLE = 32
QT = 256
NW = 32
CHUNK = 128


def _ballquery_body(nq_ref, xyz_ref, out_ref):
    b = pl.program_id(0)
    n = xyz_ref.shape[2]
    qx = nq_ref[0, :, 0:1]
    qy = nq_ref[0, :, 1:2]
    qz = nq_ref[0, :, 2:3]
    px = xyz_ref[0, 0:1, :]
    py = xyz_ref[0, 1:2, :]
    pz = xyz_ref[0, 2:3, :]
    dx = qx - px
    dy = qy - py
    dz = qz - pz
    dist2 = dx * dx + dy * dy + dz * dz
    sbit = jnp.int32(-(2**31))
    ar_x = lax.broadcasted_iota(jnp.int32, (QT, n), 1) + sbit
    n32 = jnp.int32(n)
    scores_x = jnp.where(dist2 < jnp.float32(0.1 * 0.1), ar_x, n32 + sbit)
    lane = lax.broadcasted_iota(jnp.int32, (QT, NSAMPLE), 1)

    def body(s, carry):
        m1, acc = carry
        keymin = jnp.min(scores_x - m1, axis=1, keepdims=True)
        thr = (n32 + sbit) - m1
        m2 = jnp.where(keymin <= thr, keymin + m1 + sbit, n32)
        acc2 = jnp.where(lane == s, m2, acc)
        return jnp.minimum(m2 + 1, n32), acc2

    m0 = jnp.zeros((QT, 1), jnp.int32)
    acc0 = jnp.zeros((QT, NSAMPLE), jnp.int32)
    _, acc = lax.fori_loop(0, NSAMPLE, body, (m0, acc0))
    first = acc[:, 0:1]
    first = jnp.where(first < n, first, 0)
    idx = jnp.where(acc < n, acc, first)
    out_ref[0, 0] = idx + b * n


D_TAB = 80
D_OUT = 67


def _sc_gather_body(idx_hbm, table_hbm, ctr_hbm, out_hbm,
                    idx_all, rows_a, rows_b, ctr_p, out_v,
                    sem_a, sem_b, sem_c):
    wid = lax.axis_index("s") * 2 + lax.axis_index("c")
    rows_per_worker = idx_hbm.shape[0] * CHUNK // NW
    nch = rows_per_worker // CHUNK
    qpc = CHUNK // NSAMPLE
    base = wid * rows_per_worker

    pltpu.sync_copy(idx_hbm.at[pl.ds(wid * nch, nch)], idx_all)

    def repack(rows_v, goff):
        for g in range(qpc):
            c = ctr_p[pl.ds((goff + g) * 16, 16)]
            for i in range(NSAMPLE):
                r = g * NSAMPLE + i
                out_v[pl.ds(D_OUT * r, 16)] = rows_v[r, 0:16] - c
                for v in range(1, 5):
                    out_v[pl.ds(D_OUT * r + 16 * v, 16)] = \
                        rows_v[r, 16 * v : 16 * (v + 1)]

    def pair(i, carry):
        a = 2 * i
        b = 2 * i + 1
        ha = pltpu.async_copy(table_hbm.at[idx_all.at[a]], rows_a, sem_a)
        hb = pltpu.async_copy(table_hbm.at[idx_all.at[b]], rows_b, sem_b)
        hc = pltpu.async_copy(ctr_hbm.at[wid * (nch // 2) + i], ctr_p, sem_c)
        ha.wait()
        hc.wait()
        repack(rows_a, 0)
        pltpu.sync_copy(out_v.at[pl.ds(0, CHUNK * D_OUT)],
                        out_hbm.at[pl.ds((base + a * CHUNK) * D_OUT,
                                         CHUNK * D_OUT)])
        hb.wait()
        repack(rows_b, qpc)
        pltpu.sync_copy(out_v.at[pl.ds(0, CHUNK * D_OUT)],
                        out_hbm.at[pl.ds((base + b * CHUNK) * D_OUT,
                                         CHUNK * D_OUT)])
        return carry

    lax.fori_loop(0, nch // 2, pair, 0)


def _split_pipeline(xyz, new_xyz, points):
    B, N, _ = xyz.shape
    npoint = new_xyz.shape[1]
    C = points.shape[-1]
    nt = npoint // QT

    xyz_t = jnp.transpose(xyz, (0, 2, 1))
    flat_idx = pl.pallas_call(
        _ballquery_body,
        grid=(B, nt),
        in_specs=[
            pl.BlockSpec((1, QT, 3), lambda b, t: (b, t, 0)),
            pl.BlockSpec((1, 3, N), lambda b, t: (b, 0, 0)),
        ],
        out_specs=pl.BlockSpec((1, 1, QT, NSAMPLE), lambda b, t: (b, t, 0, 0)),
        out_shape=jax.ShapeDtypeStruct((B, nt, QT, NSAMPLE), jnp.int32),
    )(new_xyz, xyz_t).reshape(-1)

    padt = jnp.zeros((B, N, D_TAB - 3 - C), jnp.float32)
    table = jnp.concatenate([xyz, points, padt], axis=-1).reshape(B * N, D_TAB)
    ctr2 = jnp.pad(new_xyz.reshape(B * npoint, 3), ((0, 0), (0, 13)))

    rows = B * npoint * NSAMPLE
    nch = rows // NW // CHUNK
    mesh = plsc.VectorSubcoreMesh(core_axis_name="c", subcore_axis_name="s")
    out = pl.kernel(
        _sc_gather_body,
        out_type=jax.ShapeDtypeStruct((rows * D_OUT,), jnp.float32),
        mesh=mesh,
        scratch_types=[
            pltpu.VMEM((nch, CHUNK), jnp.int32),
            pltpu.VMEM((CHUNK, D_TAB), jnp.float32),
            pltpu.VMEM((CHUNK, D_TAB), jnp.float32),
            pltpu.VMEM((128,), jnp.float32),
            pltpu.VMEM((CHUNK * D_OUT + 16,), jnp.float32),
            pltpu.SemaphoreType.DMA,
            pltpu.SemaphoreType.DMA,
            pltpu.SemaphoreType.DMA,
        ],
        compiler_params=pltpu.CompilerParams(use_tc_tiling_on_sc=False),
    )(flat_idx.reshape(rows // CHUNK, CHUNK), table,
      ctr2.reshape(rows // (2 * CHUNK), 128))

    return out.reshape(B, npoint, NSAMPLE, D_OUT)


def kernel(xyz, new_xyz, points):
    B = xyz.shape[0]
    h = B // 2
    lo = _split_pipeline(xyz[:h], new_xyz[:h], points[:h])
    hi = _split_pipeline(xyz[h:], new_xyz[h:], points[h:])
    return jnp.concatenate([lo, hi], axis=0)


# speedup vs baseline: 15.6902x; 1.0162x over previous
import functools

import jax
import jax.numpy as jnp
from jax import lax
from jax.experimental import pallas as pl
from jax.experimental.pallas import tpu as pltpu
from jax.experimental.pallas import tpu_sc as plsc

NSAMPLE = 32
QT = 256
NW = 32
CHUNK = 128


def _ballquery_body(nq_ref, xyz_ref, xyzr_ref, pts_ref, out_ref, tab_ref):
    b = pl.program_id(0)
    pt = pts_ref.shape[1]
    c_feat = pts_ref.shape[2]
    tab_ref[0, :, 0:3] = xyzr_ref[0]
    tab_ref[0, :, 3 : 3 + c_feat] = pts_ref[0]
    tab_ref[0, :, 3 + c_feat :] = jnp.zeros((pt, D_TAB - 3 - c_feat),
                                            jnp.float32)
    n = xyz_ref.shape[2]
    qx = nq_ref[0, :, 0:1]
    qy = nq_ref[0, :, 1:2]
    qz = nq_ref[0, :, 2:3]
    px = xyz_ref[0, 0:1, :]
    py = xyz_ref[0, 1:2, :]
    pz = xyz_ref[0, 2:3, :]
    dx = qx - px
    dy = qy - py
    dz = qz - pz
    dist2 = dx * dx + dy * dy + dz * dz
    sbit = jnp.int32(-(2**31))
    ar_x = lax.broadcasted_iota(jnp.int32, (QT, n), 1) + sbit
    n32 = jnp.int32(n)
    scores_x = jnp.where(dist2 < jnp.float32(0.1 * 0.1), ar_x, n32 + sbit)
    lane = lax.broadcasted_iota(jnp.int32, (QT, NSAMPLE), 1)

    def body(s, carry):
        m1, acc = carry
        keymin = jnp.min(scores_x - m1, axis=1, keepdims=True)
        thr = (n32 + sbit) - m1
        m2 = jnp.where(keymin <= thr, keymin + m1 + sbit, n32)
        acc2 = jnp.where(lane == s, m2, acc)
        return jnp.minimum(m2 + 1, n32), acc2

    m0 = jnp.zeros((QT, 1), jnp.int32)
    acc0 = jnp.zeros((QT, NSAMPLE), jnp.int32)
    _, acc = lax.fori_loop(0, NSAMPLE, body, (m0, acc0))
    first = acc[:, 0:1]
    first = jnp.where(first < n, first, 0)
    idx = jnp.where(acc < n, acc, first)
    out_ref[0, 0] = idx + b * n


D_TAB = 80
D_OUT = 67


def _sc_gather_body(idx_hbm, table_hbm, ctr_hbm, out_hbm,
                    idx_all, rows_a, rows_b, ctr_p, out_v,
                    sem_a, sem_b, sem_c):
    wid = lax.axis_index("s") * 2 + lax.axis_index("c")
    rows_per_worker = idx_hbm.shape[0] * CHUNK // NW
    nch = rows_per_worker // CHUNK
    qpc = CHUNK // NSAMPLE
    base = wid * rows_per_worker

    pltpu.sync_copy(idx_hbm.at[pl.ds(wid * nch, nch)], idx_all)

    def repack(rows_v, goff):
        for g in range(qpc):
            c = ctr_p[pl.ds((goff + g) * 16, 16)]
            for i in range(NSAMPLE):
                r = g * NSAMPLE + i
                out_v[pl.ds(D_OUT * r, 16)] = rows_v[r, 0:16] - c
                for v in range(1, 5):
                    out_v[pl.ds(D_OUT * r + 16 * v, 16)] = \
                        rows_v[r, 16 * v : 16 * (v + 1)]

    def pair(i, carry):
        a = 2 * i
        b = 2 * i + 1
        ha = pltpu.async_copy(table_hbm.at[idx_all.at[a]], rows_a, sem_a)
        hb = pltpu.async_copy(table_hbm.at[idx_all.at[b]], rows_b, sem_b)
        hc = pltpu.async_copy(ctr_hbm.at[wid * (nch // 2) + i], ctr_p, sem_c)
        ha.wait()
        hc.wait()
        repack(rows_a, 0)
        pltpu.sync_copy(out_v.at[pl.ds(0, CHUNK * D_OUT)],
                        out_hbm.at[pl.ds((base + a * CHUNK) * D_OUT,
                                         CHUNK * D_OUT)])
        hb.wait()
        repack(rows_b, qpc)
        pltpu.sync_copy(out_v.at[pl.ds(0, CHUNK * D_OUT)],
                        out_hbm.at[pl.ds((base + b * CHUNK) * D_OUT,
                                         CHUNK * D_OUT)])
        return carry

    lax.fori_loop(0, nch // 2, pair, 0)


def kernel(xyz, new_xyz, points):
    B, N, _ = xyz.shape
    npoint = new_xyz.shape[1]
    C = points.shape[-1]
    nt = npoint // QT
    pt = N // nt

    xyz_t = jnp.transpose(xyz, (0, 2, 1))
    flat_idx, table = pl.pallas_call(
        _ballquery_body,
        grid=(B, nt),
        in_specs=[
            pl.BlockSpec((1, QT, 3), lambda b, t: (b, t, 0)),
            pl.BlockSpec((1, 3, N), lambda b, t: (b, 0, 0)),
            pl.BlockSpec((1, pt, 3), lambda b, t: (b, t, 0)),
            pl.BlockSpec((1, pt, C), lambda b, t: (b, t, 0)),
        ],
        out_specs=[
            pl.BlockSpec((1, 1, QT, NSAMPLE), lambda b, t: (b, t, 0, 0)),
            pl.BlockSpec((1, pt, D_TAB), lambda b, t: (b, t, 0)),
        ],
        out_shape=[
            jax.ShapeDtypeStruct((B, nt, QT, NSAMPLE), jnp.int32),
            jax.ShapeDtypeStruct((B, N, D_TAB), jnp.float32),
        ],
    )(new_xyz, xyz_t, xyz, points)
    flat_idx = flat_idx.reshape(-1)
    table = table.reshape(B * N, D_TAB)
    ctr2 = jnp.pad(new_xyz.reshape(B * npoint, 3), ((0, 0), (0, 13)))

    rows = B * npoint * NSAMPLE
    nch = rows // NW // CHUNK
    mesh = plsc.VectorSubcoreMesh(core_axis_name="c", subcore_axis_name="s")
    out = pl.kernel(
        _sc_gather_body,
        out_type=jax.ShapeDtypeStruct((rows * D_OUT,), jnp.float32),
        mesh=mesh,
        scratch_types=[
            pltpu.VMEM((nch, CHUNK), jnp.int32),
            pltpu.VMEM((CHUNK, D_TAB), jnp.float32),
            pltpu.VMEM((CHUNK, D_TAB), jnp.float32),
            pltpu.VMEM((128,), jnp.float32),
            pltpu.VMEM((CHUNK * D_OUT + 16,), jnp.float32),
            pltpu.SemaphoreType.DMA,
            pltpu.SemaphoreType.DMA,
            pltpu.SemaphoreType.DMA,
        ],
        compiler_params=pltpu.CompilerParams(use_tc_tiling_on_sc=False),
    )(flat_idx.reshape(rows // CHUNK, CHUNK), table,
      ctr2.reshape(rows // (2 * CHUNK), 128))

    return out.reshape(B, npoint, NSAMPLE, D_OUT)
